# baseline (speedup 1.0000x reference)
import functools

import jax
import jax.numpy as jnp
from jax import lax
from jax.experimental import pallas as pl
from jax.experimental.pallas import tpu as pltpu
from jax.experimental.pallas import tpu_sc as plsc

_NC = 2
_NS = 16
_K = 128
_BN = 2000


def _mesh():
    return plsc.VectorSubcoreMesh(core_axis_name="c", subcore_axis_name="s")


def _make_deg(nch, n_acc):
    zper = n_acc // _NS
    h0 = -(-nch // _NC)

    @functools.partial(
        pl.kernel,
        out_type=jax.ShapeDtypeStruct((_NC, n_acc), jnp.float32),
        mesh=_mesh(),
        scratch_types=[
            pltpu.VMEM((nch, _K), jnp.int32),
            pltpu.VMEM((_K,), jnp.float32),
            pltpu.VMEM((zper,), jnp.float32),
            pltpu.VMEM_SHARED((n_acc,), jnp.float32),
        ],
    )
    def deg(col_hbm, out_hbm, col_v, ones_v, zero_v, acc_sh):
        cid = lax.axis_index("c")
        sid = lax.axis_index("s")

        def fill(i, _):
            ones_v[pl.ds(i * 16, 16)] = jnp.ones((16,), jnp.float32)
            return 0

        lax.fori_loop(0, _K // 16, fill, 0)

        def fillz(i, _):
            zero_v[pl.ds(i * 16, 16)] = jnp.zeros((16,), jnp.float32)
            return 0

        lax.fori_loop(0, zper // 16, fillz, 0)

        pltpu.sync_copy(zero_v, acc_sh.at[pl.ds(sid * zper, zper)])
        plsc.subcore_barrier()

        pltpu.sync_copy(col_hbm.at[sid], col_v)
        lo = cid * h0
        hi = jnp.minimum(lo + h0, nch)

        def body(j, _):
            pltpu.sync_copy(ones_v, acc_sh.at[col_v.at[j]], add=True)
            return 0

        lax.fori_loop(lo, hi, body, 0)
        plsc.subcore_barrier()
        pltpu.sync_copy(acc_sh.at[pl.ds(sid * zper, zper)],
                        out_hbm.at[cid, pl.ds(sid * zper, zper)])

    return deg


def _make_scatter(n, nch, n_acc, d2):
    zper = n_acc // _NS
    zrows = 4096 // d2
    nb = 4 if d2 >= 64 else 6
    words = (n * d2 + n_acc * d2
             + _NS * (2 * nch * _K + nb * _K * d2 + zrows * d2))
    sp_tbl = (words < 1_950_000) and (n % _NS == 0)

    @functools.partial(
        pl.kernel,
        out_type=jax.ShapeDtypeStruct((_NC, n_acc, d2), jnp.float32),
        mesh=_mesh(),
        compiler_params=pltpu.CompilerParams(use_tc_tiling_on_sc=False),
        scratch_types=(
            [pltpu.VMEM((nch, _K), jnp.int32),
             pltpu.VMEM((nch, _K), jnp.int32)]
            + [pltpu.VMEM((_K, d2), jnp.float32) for _ in range(nb)]
            + [pltpu.VMEM((zrows, d2), jnp.float32),
               pltpu.VMEM_SHARED((n_acc, d2), jnp.float32)]
            + ([pltpu.VMEM_SHARED((n, d2), jnp.float32)] if sp_tbl else [])
            + [pltpu.SemaphoreType.DMA for _ in range(nb + 1)]
        ),
    )
    def scat(row_hbm, col_hbm, g_hbm, out_hbm, *scr):
        row_v, col_v = scr[0], scr[1]
        bufs = scr[2:2 + nb]
        zero_v, acc_sh = scr[2 + nb], scr[3 + nb]
        nsp = 1 if sp_tbl else 0
        gsems = scr[4 + nsp + nb:4 + nsp + 2 * nb]
        ss = scr[4 + nsp + 2 * nb]
        cid = lax.axis_index("c")
        sid = lax.axis_index("s")
        if sp_tbl:
            tbl = scr[4 + nb]
            npt = n // _NS
            pltpu.sync_copy(g_hbm.at[cid, pl.ds(sid * npt, npt)],
                            tbl.at[pl.ds(sid * npt, npt)])
        else:
            tbl = g_hbm.at[cid]

        def fillz(r, _):
            for cc in range(d2 // 16):
                zero_v[r, pl.ds(cc * 16, 16)] = jnp.zeros((16,), jnp.float32)
            return 0

        lax.fori_loop(0, zrows, fillz, 0)

        def zcopy(t, _):
            pltpu.sync_copy(
                zero_v, acc_sh.at[pl.ds(sid * zper + t * zrows, zrows)])
            return 0

        lax.fori_loop(0, zper // zrows, zcopy, 0)
        plsc.subcore_barrier()

        pltpu.sync_copy(row_hbm.at[sid], row_v)
        pltpu.sync_copy(col_hbm.at[sid], col_v)

        def gath(j, b):
            pltpu.async_copy(tbl.at[row_v.at[j]], bufs[b], gsems[b])

        def gwait(j, b):
            pltpu.make_async_copy(tbl.at[row_v.at[j]], bufs[b],
                                  gsems[b]).wait()

        def sc_issue(j, b):
            pltpu.async_copy(bufs[b], acc_sh.at[col_v.at[j]], ss, add=True)

        def sc_wait(j, b):
            pltpu.make_async_copy(bufs[b], acc_sh.at[col_v.at[j]],
                                  ss).wait()

        if nch < 2 * nb:
            for j in range(nch):
                gath(j, j % nb)
                gwait(j, j % nb)
                sc_issue(j, j % nb)
                sc_wait(j, j % nb)
        else:
            for j in range(nb - 1):
                gath(j, j)
            gwait(0, 0)
            sc_issue(0, 0)
            gath(nb - 1, nb - 1)
            nin = nch - nb
            u, r = divmod(nin, nb)

            def step(j, i):
                sc_wait(j - 1, (i - 1) % nb)
                gath(j + nb - 1, (i - 1) % nb)
                gwait(j, i)
                sc_issue(j, i)

            def body(t, _):
                j = 1 + nb * t
                for i in range(nb):
                    step(j + i, (1 + i) % nb)
                return 0

            lax.fori_loop(0, u, body, 0)
            jb = 1 + nb * u
            for i in range(r):
                step(jb + i, (jb + i) % nb)
            for j in range(nch - nb + 1, nch):
                sc_wait(j - 1, (j - 1) % nb)
                gwait(j, j % nb)
                sc_issue(j, j % nb)
            sc_wait(nch - 1, (nch - 1) % nb)

        plsc.subcore_barrier()
        pltpu.sync_copy(acc_sh.at[pl.ds(sid * zper, zper)],
                        out_hbm.at[cid, pl.ds(sid * zper, zper)])

    return scat


def _tc_first(x, w1, cnt):
    n, f_in = x.shape
    h = w1.shape[1]
    d2 = h // 2

    def body(x_ref, w_ref, c_ref, o_ref):
        c = c_ref[:, 0] + c_ref[:, 1] + 1.0
        dis = lax.rsqrt(c)
        g = jnp.dot(x_ref[...], w_ref[...], preferred_element_type=jnp.float32)
        g = g * dis[:, None]
        o_ref[0] = g[:, :d2]
        o_ref[1] = g[:, d2:]

    return pl.pallas_call(
        body,
        grid=(n // _BN,),
        in_specs=[
            pl.BlockSpec((_BN, f_in), lambda i: (i, 0)),
            pl.BlockSpec((f_in, h), lambda i: (0, 0)),
            pl.BlockSpec((_BN, 2), lambda i: (i, 0)),
        ],
        out_specs=pl.BlockSpec((2, _BN, d2), lambda i: (0, i, 0)),
        out_shape=jax.ShapeDtypeStruct((2, n, d2), jnp.float32),
    )(x, w1, cnt)


def _tc_mid(p, g1, cnt, w2):
    n = g1.shape[1]
    h = 2 * g1.shape[2]
    c = w2.shape[1]
    d2 = c // 2

    def body(p_ref, g_ref, c_ref, w_ref, o_ref):
        cn = c_ref[:, 0] + c_ref[:, 1] + 1.0
        dis = lax.rsqrt(cn)
        s = jnp.concatenate([p_ref[0] + g_ref[0], p_ref[1] + g_ref[1]],
                            axis=1)
        h1 = jnp.maximum(s * dis[:, None], 0.0)
        g2 = jnp.dot(h1, w_ref[...],
                     preferred_element_type=jnp.float32) * dis[:, None]
        o_ref[0] = g2[:, :d2]
        o_ref[1] = g2[:, d2:]

    return pl.pallas_call(
        body,
        grid=(n // _BN,),
        in_specs=[
            pl.BlockSpec((2, _BN, h // 2), lambda i: (0, i, 0)),
            pl.BlockSpec((2, _BN, h // 2), lambda i: (0, i, 0)),
            pl.BlockSpec((_BN, 2), lambda i: (i, 0)),
            pl.BlockSpec((h, c), lambda i: (0, 0)),
        ],
        out_specs=pl.BlockSpec((2, _BN, d2), lambda i: (0, i, 0)),
        out_shape=jax.ShapeDtypeStruct((2, n, d2), jnp.float32),
    )(p, g1, cnt, w2)


def _tc_last(p, g2, cnt):
    n = g2.shape[1]
    c = 2 * g2.shape[2]

    def body(p_ref, g_ref, c_ref, o_ref):
        cn = c_ref[:, 0] + c_ref[:, 1] + 1.0
        dis = lax.rsqrt(cn)
        s = jnp.concatenate([p_ref[0] + g_ref[0], p_ref[1] + g_ref[1]],
                            axis=1)
        o_ref[...] = jnp.maximum(s * dis[:, None], 0.0)

    return pl.pallas_call(
        body,
        grid=(n // _BN,),
        in_specs=[
            pl.BlockSpec((2, _BN, c // 2), lambda i: (0, i, 0)),
            pl.BlockSpec((2, _BN, c // 2), lambda i: (0, i, 0)),
            pl.BlockSpec((_BN, 2), lambda i: (i, 0)),
        ],
        out_specs=pl.BlockSpec((_BN, c), lambda i: (i, 0)),
        out_shape=jax.ShapeDtypeStruct((n, c), jnp.float32),
    )(p, g2, cnt)


def kernel(x, edge_index, W1, W2):
    n, f_in = x.shape
    e = edge_index.shape[1]

    nch = -(-e // (_NS * _K))
    e_pad = _NS * _K * nch
    pad = e_pad - e
    zper = -(-(n + 8) // (_NS * 64)) * 64
    n_acc = zper * _NS

    row = edge_index[0].astype(jnp.int32)
    col = edge_index[1].astype(jnp.int32)
    row3 = jnp.concatenate(
        [row, jnp.zeros((pad,), jnp.int32)]).reshape(_NS, nch, _K)
    col3 = jnp.concatenate(
        [col, jnp.full((pad,), n, jnp.int32)]).reshape(_NS, nch, _K)

    cnt = _make_deg(nch, n_acc)(col3)
    cnt2 = cnt.T

    g1 = _tc_first(x, W1, cnt2)
    p1 = _make_scatter(n, nch, n_acc, g1.shape[2])(row3, col3, g1)
    g2 = _tc_mid(p1, g1, cnt2, W2)
    p2 = _make_scatter(n, nch, n_acc, g2.shape[2])(row3, col3, g2)
    out = _tc_last(p2, g2, cnt2)
    return out


# speedup vs baseline: 1.0279x; 1.0279x over previous
import functools

import jax
import jax.numpy as jnp
from jax import lax
from jax.experimental import pallas as pl
from jax.experimental.pallas import tpu as pltpu
from jax.experimental.pallas import tpu_sc as plsc

_NC = 2
_NS = 16
_K = 128
_BN = 5000


def _mesh():
    return plsc.VectorSubcoreMesh(core_axis_name="c", subcore_axis_name="s")


def _make_deg(nch, n_acc):
    zper = n_acc // _NS
    h0 = -(-nch // _NC)

    @functools.partial(
        pl.kernel,
        out_type=jax.ShapeDtypeStruct((_NC, n_acc), jnp.float32),
        mesh=_mesh(),
        scratch_types=[
            pltpu.VMEM((nch, _K), jnp.int32),
            pltpu.VMEM((_K,), jnp.float32),
            pltpu.VMEM((zper,), jnp.float32),
            pltpu.VMEM_SHARED((n_acc,), jnp.float32),
        ],
    )
    def deg(col_hbm, out_hbm, col_v, ones_v, zero_v, acc_sh):
        cid = lax.axis_index("c")
        sid = lax.axis_index("s")

        def fill(i, _):
            ones_v[pl.ds(i * 16, 16)] = jnp.ones((16,), jnp.float32)
            return 0

        lax.fori_loop(0, _K // 16, fill, 0)

        def fillz(i, _):
            zero_v[pl.ds(i * 16, 16)] = jnp.zeros((16,), jnp.float32)
            return 0

        lax.fori_loop(0, zper // 16, fillz, 0)

        pltpu.sync_copy(zero_v, acc_sh.at[pl.ds(sid * zper, zper)])
        plsc.subcore_barrier()

        pltpu.sync_copy(col_hbm.at[sid], col_v)
        lo = cid * h0
        hi = jnp.minimum(lo + h0, nch)

        def body(j, _):
            pltpu.sync_copy(ones_v, acc_sh.at[col_v.at[j]], add=True)
            return 0

        lax.fori_loop(lo, hi, body, 0)
        plsc.subcore_barrier()
        pltpu.sync_copy(acc_sh.at[pl.ds(sid * zper, zper)],
                        out_hbm.at[cid, pl.ds(sid * zper, zper)])

    return deg


def _make_scatter(n, nch, n_acc, d2):
    zper = n_acc // _NS
    zrows = 4096 // d2
    nb = 4 if d2 >= 64 else 6
    del n

    @functools.partial(
        pl.kernel,
        out_type=jax.ShapeDtypeStruct((_NC, n_acc, d2), jnp.float32),
        mesh=_mesh(),
        compiler_params=pltpu.CompilerParams(use_tc_tiling_on_sc=False),
        scratch_types=(
            [pltpu.VMEM((nch, _K), jnp.int32),
             pltpu.VMEM((nch, _K), jnp.int32)]
            + [pltpu.VMEM((_K, d2), jnp.float32) for _ in range(nb)]
            + [pltpu.VMEM((zrows, d2), jnp.float32),
               pltpu.VMEM_SHARED((n_acc, d2), jnp.float32)]
            + [pltpu.SemaphoreType.DMA for _ in range(nb + 1)]
        ),
    )
    def scat(row_hbm, col_hbm, g_hbm, out_hbm, *scr):
        row_v, col_v = scr[0], scr[1]
        bufs = scr[2:2 + nb]
        zero_v, acc_sh = scr[2 + nb], scr[3 + nb]
        gsems = scr[4 + nb:4 + 2 * nb]
        ss = scr[4 + 2 * nb]
        cid = lax.axis_index("c")
        sid = lax.axis_index("s")
        tbl = g_hbm.at[cid]

        def fillz(r, _):
            for cc in range(d2 // 16):
                zero_v[r, pl.ds(cc * 16, 16)] = jnp.zeros((16,), jnp.float32)
            return 0

        lax.fori_loop(0, zrows, fillz, 0)

        def zcopy(t, _):
            pltpu.sync_copy(
                zero_v, acc_sh.at[pl.ds(sid * zper + t * zrows, zrows)])
            return 0

        lax.fori_loop(0, zper // zrows, zcopy, 0)
        plsc.subcore_barrier()

        pltpu.sync_copy(row_hbm.at[sid], row_v)
        pltpu.sync_copy(col_hbm.at[sid], col_v)

        def gath(j, b):
            pltpu.async_copy(tbl.at[row_v.at[j]], bufs[b], gsems[b])

        def gwait(j, b):
            pltpu.make_async_copy(tbl.at[row_v.at[j]], bufs[b],
                                  gsems[b]).wait()

        def sc_issue(j, b):
            pltpu.async_copy(bufs[b], acc_sh.at[col_v.at[j]], ss, add=True)

        def sc_wait(j, b):
            pltpu.make_async_copy(bufs[b], acc_sh.at[col_v.at[j]],
                                  ss).wait()

        if nch < 2 * nb:
            for j in range(nch):
                gath(j, j % nb)
                gwait(j, j % nb)
                sc_issue(j, j % nb)
                sc_wait(j, j % nb)
        else:
            for j in range(nb - 1):
                gath(j, j)
            gwait(0, 0)
            sc_issue(0, 0)
            gath(nb - 1, nb - 1)
            nin = nch - nb
            u, r = divmod(nin, nb)

            def step(j, i):
                sc_wait(j - 1, (i - 1) % nb)
                gath(j + nb - 1, (i - 1) % nb)
                gwait(j, i)
                sc_issue(j, i)

            def body(t, _):
                j = 1 + nb * t
                for i in range(nb):
                    step(j + i, (1 + i) % nb)
                return 0

            lax.fori_loop(0, u, body, 0)
            jb = 1 + nb * u
            for i in range(r):
                step(jb + i, (jb + i) % nb)
            for j in range(nch - nb + 1, nch):
                sc_wait(j - 1, (j - 1) % nb)
                gwait(j, j % nb)
                sc_issue(j, j % nb)
            sc_wait(nch - 1, (nch - 1) % nb)

        plsc.subcore_barrier()
        pltpu.sync_copy(acc_sh.at[pl.ds(sid * zper, zper)],
                        out_hbm.at[cid, pl.ds(sid * zper, zper)])

    return scat


def _tc_first(x, w1, cnt):
    n, f_in = x.shape
    h = w1.shape[1]
    d2 = h // 2

    def body(x_ref, w_ref, c_ref, o_ref):
        c = c_ref[:, 0] + c_ref[:, 1] + 1.0
        dis = lax.rsqrt(c)
        g = jnp.dot(x_ref[...], w_ref[...], preferred_element_type=jnp.float32)
        g = g * dis[:, None]
        o_ref[0] = g[:, :d2]
        o_ref[1] = g[:, d2:]

    return pl.pallas_call(
        body,
        grid=(n // _BN,),
        in_specs=[
            pl.BlockSpec((_BN, f_in), lambda i: (i, 0)),
            pl.BlockSpec((f_in, h), lambda i: (0, 0)),
            pl.BlockSpec((_BN, 2), lambda i: (i, 0)),
        ],
        out_specs=pl.BlockSpec((2, _BN, d2), lambda i: (0, i, 0)),
        out_shape=jax.ShapeDtypeStruct((2, n, d2), jnp.float32),
    )(x, w1, cnt)


def _tc_mid(p, g1, cnt, w2):
    n = g1.shape[1]
    h = 2 * g1.shape[2]
    c = w2.shape[1]
    d2 = c // 2

    def body(p_ref, g_ref, c_ref, w_ref, o_ref):
        cn = c_ref[:, 0] + c_ref[:, 1] + 1.0
        dis = lax.rsqrt(cn)
        s = jnp.concatenate([p_ref[0] + g_ref[0], p_ref[1] + g_ref[1]],
                            axis=1)
        h1 = jnp.maximum(s * dis[:, None], 0.0)
        g2 = jnp.dot(h1, w_ref[...],
                     preferred_element_type=jnp.float32) * dis[:, None]
        o_ref[0] = g2[:, :d2]
        o_ref[1] = g2[:, d2:]

    return pl.pallas_call(
        body,
        grid=(n // _BN,),
        in_specs=[
            pl.BlockSpec((2, _BN, h // 2), lambda i: (0, i, 0)),
            pl.BlockSpec((2, _BN, h // 2), lambda i: (0, i, 0)),
            pl.BlockSpec((_BN, 2), lambda i: (i, 0)),
            pl.BlockSpec((h, c), lambda i: (0, 0)),
        ],
        out_specs=pl.BlockSpec((2, _BN, d2), lambda i: (0, i, 0)),
        out_shape=jax.ShapeDtypeStruct((2, n, d2), jnp.float32),
    )(p, g1, cnt, w2)


def _tc_last(p, g2, cnt):
    n = g2.shape[1]
    c = 2 * g2.shape[2]

    def body(p_ref, g_ref, c_ref, o_ref):
        cn = c_ref[:, 0] + c_ref[:, 1] + 1.0
        dis = lax.rsqrt(cn)
        s = jnp.concatenate([p_ref[0] + g_ref[0], p_ref[1] + g_ref[1]],
                            axis=1)
        o_ref[...] = jnp.maximum(s * dis[:, None], 0.0)

    return pl.pallas_call(
        body,
        grid=(n // _BN,),
        in_specs=[
            pl.BlockSpec((2, _BN, c // 2), lambda i: (0, i, 0)),
            pl.BlockSpec((2, _BN, c // 2), lambda i: (0, i, 0)),
            pl.BlockSpec((_BN, 2), lambda i: (i, 0)),
        ],
        out_specs=pl.BlockSpec((_BN, c), lambda i: (i, 0)),
        out_shape=jax.ShapeDtypeStruct((n, c), jnp.float32),
    )(p, g2, cnt)


def kernel(x, edge_index, W1, W2):
    n, f_in = x.shape
    e = edge_index.shape[1]

    nch = -(-e // (_NS * _K))
    e_pad = _NS * _K * nch
    pad = e_pad - e
    zper = -(-(n + 8) // (_NS * 64)) * 64
    n_acc = zper * _NS

    row = edge_index[0].astype(jnp.int32)
    col = edge_index[1].astype(jnp.int32)
    row3 = jnp.concatenate(
        [row, jnp.zeros((pad,), jnp.int32)]).reshape(_NS, nch, _K)
    col3 = jnp.concatenate(
        [col, jnp.full((pad,), n, jnp.int32)]).reshape(_NS, nch, _K)

    cnt = _make_deg(nch, n_acc)(col3)
    cnt2 = cnt.T

    g1 = _tc_first(x, W1, cnt2)
    p1 = _make_scatter(n, nch, n_acc, g1.shape[2])(row3, col3, g1)
    g2 = _tc_mid(p1, g1, cnt2, W2)
    p2 = _make_scatter(n, nch, n_acc, g2.shape[2])(row3, col3, g2)
    out = _tc_last(p2, g2, cnt2)
    return out
